# Initial kernel scaffold; baseline (speedup 1.0000x reference)
import jax
import jax.numpy as jnp
from jax.experimental import pallas as pl


def kernel(inputs):
    raise NotImplementedError("write your pallas kernel here")


# speedup vs baseline: 3.9306x; 3.9306x over previous
import functools

import jax
import jax.numpy as jnp
from jax import lax
from jax.experimental import pallas as pl
from jax.experimental.pallas import tpu as pltpu
from jax.experimental.pallas import tpu_sc as plsc

_BINS = [-3.0, -2.8, -2.6, -2.4, -2.2, -2.0, -1.8, -1.6, -1.4, -1.2, -1.0,
         -0.8, -0.6, -0.4, -0.2, 0.0, 0.2, 0.4, 0.6, 0.8, 1.0, 1.2, 1.4,
         1.6, 1.8, 2.0, 2.2, 2.4, 2.6, 2.8, 3.0]

_N = 16384 * 4096
_NC = 2
_NS = 16
_NW = _NC * _NS
_PER_W = _N // _NW
_CHUNK = 16384
_NCHUNK = _PER_W // _CHUNK


def _body(x_hbm, bins_hbm, out_hbm, bins_v, inbuf, outbuf):
    wid = lax.axis_index("s") * _NC + lax.axis_index("c")
    base_w = wid * _PER_W
    pltpu.sync_copy(bins_hbm, bins_v)

    def chunk_body(ci, carry):
        base = base_w + ci * _CHUNK
        pltpu.sync_copy(x_hbm.at[pl.ds(base, _CHUNK)], inbuf)

        def vec_body(j, carry2):
            v = inbuf[pl.ds(j * 16, 16)]
            t = v * 5.0 + 15.0
            t = jnp.minimum(jnp.maximum(t, 0.0), 30.0)
            k = (t + 0.5).astype(jnp.int32)
            b = plsc.load_gather(bins_v, [k])
            cnt = k + jnp.where(b <= v, 1, 0)
            outbuf[pl.ds(j * 16, 16)] = cnt
            return carry2

        lax.fori_loop(0, _CHUNK // 16, vec_body, 0)
        pltpu.sync_copy(outbuf, out_hbm.at[pl.ds(base, _CHUNK)])
        return carry

    lax.fori_loop(0, _NCHUNK, chunk_body, 0)


@jax.jit
def kernel(inputs):
    x = inputs.reshape(-1)
    bins = jnp.asarray(_BINS + [3.0], dtype=jnp.float32)
    mesh = plsc.VectorSubcoreMesh(core_axis_name="c", subcore_axis_name="s")
    run = functools.partial(
        pl.kernel,
        out_type=jax.ShapeDtypeStruct((_N,), jnp.int32),
        mesh=mesh,
        compiler_params=pltpu.CompilerParams(needs_layout_passes=False),
        scratch_types=[
            pltpu.VMEM((32,), jnp.float32),
            pltpu.VMEM((_CHUNK,), jnp.float32),
            pltpu.VMEM((_CHUNK,), jnp.int32),
        ],
    )(_body)
    out = run(x, bins)
    return out.reshape(inputs.shape)


# speedup vs baseline: 8.2660x; 2.1030x over previous
import functools

import jax
import jax.numpy as jnp
from jax import lax
from jax.experimental import pallas as pl
from jax.experimental.pallas import tpu as pltpu
from jax.experimental.pallas import tpu_sc as plsc

_BINS = [-3.0, -2.8, -2.6, -2.4, -2.2, -2.0, -1.8, -1.6, -1.4, -1.2, -1.0,
         -0.8, -0.6, -0.4, -0.2, 0.0, 0.2, 0.4, 0.6, 0.8, 1.0, 1.2, 1.4,
         1.6, 1.8, 2.0, 2.2, 2.4, 2.6, 2.8, 3.0]

_N = 16384 * 4096
_NC = 2
_NS = 16
_NW = _NC * _NS
_PER_W = _N // _NW
_CHUNK = 16384
_NCHUNK = _PER_W // _CHUNK


def _compute(inbuf, outbuf, bins_v):
    @plsc.parallel_loop(0, _CHUNK, step=16, unroll=8)
    def vec_body(i):
        v = inbuf[pl.ds(i, 16)]
        t = v * 5.0 + 15.0
        t = jnp.minimum(jnp.maximum(t, 0.0), 30.0)
        k = (t + 0.5).astype(jnp.int32)
        b = plsc.load_gather(bins_v, [k])
        outbuf[pl.ds(i, 16)] = k + jnp.where(b <= v, 1, 0)


def _body(x_hbm, bins_hbm, out_hbm, bins_v, in0, in1, out0, out1,
          si0, si1, so0, so1):
    wid = lax.axis_index("s") * _NC + lax.axis_index("c")
    base_w = wid * _PER_W
    pltpu.sync_copy(bins_hbm, bins_v)
    ins, outs = (in0, in1), (out0, out1)
    sis, sos = (si0, si1), (so0, so1)

    pltpu.async_copy(x_hbm.at[pl.ds(base_w, _CHUNK)], in0, si0)
    pltpu.async_copy(x_hbm.at[pl.ds(base_w + _CHUNK, _CHUNK)], in1, si1)

    @pl.loop(0, _NCHUNK // 2)
    def group(g):
        for b in range(2):
            ci = g * 2 + b
            base = base_w + ci * _CHUNK
            pltpu.make_async_copy(
                x_hbm.at[pl.ds(base, _CHUNK)], ins[b], sis[b]).wait()

            @pl.when(ci >= 2)
            def _wait_prev_out():
                pltpu.make_async_copy(
                    outs[b], out_hbm.at[pl.ds(base - 2 * _CHUNK, _CHUNK)],
                    sos[b]).wait()

            _compute(ins[b], outs[b], bins_v)

            @pl.when(ci + 2 < _NCHUNK)
            def _start_next_in():
                pltpu.async_copy(
                    x_hbm.at[pl.ds(base + 2 * _CHUNK, _CHUNK)], ins[b], sis[b])

            pltpu.async_copy(outs[b], out_hbm.at[pl.ds(base, _CHUNK)], sos[b])

    for b in range(2):
        ci = _NCHUNK - 2 + b
        pltpu.make_async_copy(
            outs[b], out_hbm.at[pl.ds(base_w + ci * _CHUNK, _CHUNK)],
            sos[b]).wait()


@jax.jit
def kernel(inputs):
    x = inputs.reshape(-1)
    bins = jnp.asarray(_BINS + [3.0], dtype=jnp.float32)
    mesh = plsc.VectorSubcoreMesh(core_axis_name="c", subcore_axis_name="s")
    run = functools.partial(
        pl.kernel,
        out_type=jax.ShapeDtypeStruct((_N,), jnp.int32),
        mesh=mesh,
        compiler_params=pltpu.CompilerParams(needs_layout_passes=False),
        scratch_types=[
            pltpu.VMEM((32,), jnp.float32),
            pltpu.VMEM((_CHUNK,), jnp.float32),
            pltpu.VMEM((_CHUNK,), jnp.float32),
            pltpu.VMEM((_CHUNK,), jnp.int32),
            pltpu.VMEM((_CHUNK,), jnp.int32),
            pltpu.SemaphoreType.DMA,
            pltpu.SemaphoreType.DMA,
            pltpu.SemaphoreType.DMA,
            pltpu.SemaphoreType.DMA,
        ],
    )(_body)
    out = run(x, bins)
    return out.reshape(inputs.shape)


# speedup vs baseline: 20.5985x; 2.4920x over previous
import functools

import jax
import jax.numpy as jnp
from jax import lax
from jax.experimental import pallas as pl
from jax.experimental.pallas import tpu as pltpu
from jax.experimental.pallas import tpu_sc as plsc

_BINS = [-3.0, -2.8, -2.6, -2.4, -2.2, -2.0, -1.8, -1.6, -1.4, -1.2, -1.0,
         -0.8, -0.6, -0.4, -0.2, 0.0, 0.2, 0.4, 0.6, 0.8, 1.0, 1.2, 1.4,
         1.6, 1.8, 2.0, 2.2, 2.4, 2.6, 2.8, 3.0]

_ROWS = 16384
_COLS = 4096
_NC = 2
_NS = 16
_NW = _NC * _NS
_RW = _ROWS // _NW
_CR = 8
_CC = 2048
_NG = _RW // _CR


def _compute(inbuf, outbuf, bins_v):
    for r in range(_CR):
        @plsc.parallel_loop(0, _CC, step=16, unroll=8)
        def vec_body(c):
            v = inbuf[r, pl.ds(c, 16)]
            t = v * 5.0 + 15.5
            t = jnp.minimum(jnp.maximum(t, 0.5), 30.5)
            k = t.astype(jnp.int32)
            b = plsc.load_gather(bins_v, [k])
            outbuf[r, pl.ds(c, 16)] = k + jnp.where(b <= v, 1, 0)


def _body(x_hbm, bins_hbm, out_hbm, bins_v, in0, in1, out0, out1,
          si0, si1, so0, so1):
    wid = lax.axis_index("s") * _NC + lax.axis_index("c")
    row_w = wid * _RW
    pltpu.sync_copy(bins_hbm, bins_v)
    ins, outs = (in0, in1), (out0, out1)
    sis, sos = (si0, si1), (so0, so1)

    for b in range(2):
        pltpu.async_copy(
            x_hbm.at[pl.ds(row_w, _CR), pl.ds(b * _CC, _CC)], ins[b], sis[b])

    @pl.loop(0, _NG)
    def group(g):
        r0 = row_w + g * _CR
        for b in range(2):
            c0 = b * _CC
            pltpu.make_async_copy(
                x_hbm.at[pl.ds(r0, _CR), pl.ds(c0, _CC)], ins[b],
                sis[b]).wait()

            @pl.when(g >= 1)
            def _wait_prev_out():
                pltpu.make_async_copy(
                    outs[b], out_hbm.at[pl.ds(r0 - _CR, _CR), pl.ds(c0, _CC)],
                    sos[b]).wait()

            _compute(ins[b], outs[b], bins_v)

            @pl.when(g + 1 < _NG)
            def _start_next_in():
                pltpu.async_copy(
                    x_hbm.at[pl.ds(r0 + _CR, _CR), pl.ds(c0, _CC)],
                    ins[b], sis[b])

            pltpu.async_copy(
                outs[b], out_hbm.at[pl.ds(r0, _CR), pl.ds(c0, _CC)], sos[b])

    last_r0 = row_w + (_NG - 1) * _CR
    for b in range(2):
        pltpu.make_async_copy(
            outs[b], out_hbm.at[pl.ds(last_r0, _CR), pl.ds(b * _CC, _CC)],
            sos[b]).wait()


@jax.jit
def kernel(inputs):
    bins = jnp.asarray(_BINS + [3.0], dtype=jnp.float32)
    mesh = plsc.VectorSubcoreMesh(core_axis_name="c", subcore_axis_name="s")
    run = functools.partial(
        pl.kernel,
        out_type=jax.ShapeDtypeStruct((_ROWS, _COLS), jnp.int32),
        mesh=mesh,
        compiler_params=pltpu.CompilerParams(needs_layout_passes=False),
        scratch_types=[
            pltpu.VMEM((32,), jnp.float32),
            pltpu.VMEM((_CR, _CC), jnp.float32),
            pltpu.VMEM((_CR, _CC), jnp.float32),
            pltpu.VMEM((_CR, _CC), jnp.int32),
            pltpu.VMEM((_CR, _CC), jnp.int32),
            pltpu.SemaphoreType.DMA,
            pltpu.SemaphoreType.DMA,
            pltpu.SemaphoreType.DMA,
            pltpu.SemaphoreType.DMA,
        ],
    )(_body)
    return run(inputs, bins)
